# baseline (speedup 1.0000x reference)
import jax
import jax.numpy as jnp
from jax.experimental import pallas as pl


def _spmm(rows, cols, vals, x, n_out):
    return jnp.zeros((n_out, x.shape[1]), x.dtype).at[rows].add(vals[:, None] * x[cols])


def _final_linear_kernel(x_ref, w_ref, b_ref, o_ref):
    o_ref[...] = jax.nn.sigmoid(x_ref[...]) @ w_ref[...] + b_ref[0, 0]


def kernel(x0, x1, x2, b1_row, b1_col, b1_val, b2_row, b2_col, b2_val,
           a0_row, a0_col, a0_val, a1_row, a1_col, a1_val, a2_row, a2_col,
           a2_val, params):
    N0, N1 = x0.shape[0], x1.shape[0]
    C = x0.shape[1]

    m0 = _spmm(a0_row, a0_col, a0_val, x0 @ params['W_same_0_0'], N0)
    m0 = m0 + _spmm(b1_row, b1_col, b1_val, x1 @ params['W_htl_0_0'], N0)
    m1 = _spmm(a1_row, a1_col, a1_val, x1 @ params['W_same_0_1'], N1)
    m1 = m1 + _spmm(b2_row, b2_col, b2_val, x2 @ params['W_htl_0_1'], N1)
    m1 = m1 + _spmm(b1_col, b1_row, b1_val, x0 @ params['W_lth_0_1'], N1)
    f0, f1 = jax.nn.sigmoid(m0), jax.nn.sigmoid(m1)

    m0 = _spmm(a0_row, a0_col, a0_val, f0 @ params['W_same_1_0'], N0)
    m0 = m0 + _spmm(b1_row, b1_col, b1_val, f1 @ params['W_htl_1_0'], N0)

    out = pl.pallas_call(
        _final_linear_kernel,
        grid=(10,),
        in_specs=[
            pl.BlockSpec((N0 // 10, C), lambda i: (i, 0)),
            pl.BlockSpec((C, 1), lambda i: (0, 0)),
            pl.BlockSpec((1, 1), lambda i: (0, 0)),
        ],
        out_specs=pl.BlockSpec((N0 // 10, 1), lambda i: (i, 0)),
        out_shape=jax.ShapeDtypeStruct((N0, 1), jnp.float32),
    )(m0, params['W_lin'], params['b_lin'].reshape(1, 1))
    return out.reshape(N0)


# speedup vs baseline: 1.4204x; 1.4204x over previous
import jax
import jax.numpy as jnp
from jax import lax
from jax.experimental import pallas as pl
from jax.experimental.pallas import tpu as pltpu
from jax.experimental.pallas import tpu_sc as plsc

_L = 16
_NT = 16
_W = 2048
_GRP = _W // _L
_RB = 128
_BUF = _W + _RB + _L
_ZR = 40
_C = 128


def _pad_task(r, c, v, n_dest):
    n = r.shape[0]
    q = _NT * _W
    m = ((n + q - 1) // q) * q - n
    if m:
        r = jnp.concatenate([r.astype(jnp.int32),
                             jnp.full((m,), n_dest, jnp.int32)])
        c = jnp.concatenate([c.astype(jnp.int32), jnp.zeros((m,), jnp.int32)])
        v = jnp.concatenate([v, jnp.zeros((m,), v.dtype)])
    else:
        r = r.astype(jnp.int32)
        c = c.astype(jnp.int32)
    return r, c, v


def _gat16(x, idx):
    dn = lax.GatherDimensionNumbers(offset_dims=(), collapsed_slice_dims=(0,),
                                    start_index_map=(0,))
    return lax.gather(x, idx[:, None], dn, slice_sizes=(1,),
                      mode=lax.GatherScatterMode.PROMISE_IN_BOUNDS)


def _build_sc_body(phases, n_srcs, n_tasks):
    def body(*refs):
        srcs = refs[:n_srcs]
        tr = refs[n_srcs:n_srcs + 3 * n_tasks]
        outs = refs[n_srcs + 3 * n_tasks:-12]
        (spmem, rbuf, cbuf, vbuf, rlocm, cm, vm,
         rlocf, cf, rows, zrows, gsem) = refs[-12:]

        cid = lax.axis_index("c")
        sid = lax.axis_index("s")

        def zz(i, _):
            for k in range(8):
                zrows[i, pl.ds(k * _L, _L)] = jnp.zeros((_L,), jnp.float32)
            return 0
        lax.fori_loop(0, _ZR, zz, 0)

        def zm(i, _):
            rlocm[pl.ds(i * _L, _L)] = jnp.zeros((_L,), jnp.int32)
            cm[pl.ds(i * _L, _L)] = jnp.zeros((_L,), jnp.int32)
            vm[pl.ds(i * _L, _L)] = jnp.zeros((_L,), jnp.float32)
            return 0
        lax.fori_loop(0, _BUF // _L, zm, 0)

        def flush(src, off):
            for k in range(_RB // _L):
                rlocf[pl.ds(k * _L, _L)] = rlocm[pl.ds(off + k * _L, _L)]
                cf[pl.ds(k * _L, _L)] = cm[pl.ds(off + k * _L, _L)]
            pltpu.async_copy(src.at[cf], rows, gsem).wait()

            def srow(i, _):
                for jj in range(4):
                    jr = i * 4 + jj
                    vj = vm[pl.ds(off + jr, _L)][0]
                    for kk in range(_C // _L):
                        rows[jr, pl.ds(kk * _L, _L)] = (
                            rows[jr, pl.ds(kk * _L, _L)] * vj)
                return 0
            lax.fori_loop(0, _RB // 4, srow, 0)
            pltpu.sync_copy(rows, spmem.at[rlocf], add=True)

        def run_task(src, r_h, c_h, v_h, base, ch):
            per_tile = r_h.shape[0] // _NT
            n_win = per_tile // _W

            def win(w, cnt):
                start = sid * per_tile + w * _W
                pltpu.sync_copy(r_h.at[pl.ds(start, _W)], rbuf)
                pltpu.sync_copy(c_h.at[pl.ds(start, _W)], cbuf)
                pltpu.sync_copy(v_h.at[pl.ds(start, _W)], vbuf)

                iota = lax.iota(jnp.int32, _L)

                def grp(g, cnt):
                    rv = rbuf[pl.ds(g * _L, _L)]
                    m = (rv >= base) & (rv < base + ch)
                    p = jnp.where(m, 1, 0)
                    for s in (1, 2, 4, 8):
                        w = _gat16(p, jnp.maximum(iota - s, 0))
                        p = p + jnp.where(iota >= s, w, 0)
                    total = p[15]
                    tgt = iota + 1
                    lo = jnp.zeros((_L,), jnp.int32)
                    hi = jnp.full((_L,), _L - 1, jnp.int32)
                    for _ in range(4):
                        mid = lax.shift_right_logical(lo + hi, 1)
                        ge = _gat16(p, mid) >= tgt
                        hi = jnp.where(ge, mid, hi)
                        lo = jnp.where(ge, lo, mid + 1)
                    rlocm[pl.ds(cnt, _L)] = jnp.clip(
                        _gat16(rv, hi) - base, 0, ch - 1)
                    cm[pl.ds(cnt, _L)] = _gat16(cbuf[pl.ds(g * _L, _L)], hi)
                    vm[pl.ds(cnt, _L)] = _gat16(vbuf[pl.ds(g * _L, _L)], hi)
                    return cnt + total
                cnt = lax.fori_loop(0, _GRP, grp, cnt)

                nfull = cnt // _RB

                def fb(i, _):
                    flush(src, i * _RB)
                    return 0
                lax.fori_loop(0, nfull, fb, 0)

                rem = cnt - nfull * _RB
                for k in range(_RB // _L):
                    @pl.when(k * _L < rem)
                    def _(k=k):
                        o = nfull * _RB + k * _L
                        rlocm[pl.ds(k * _L, _L)] = rlocm[pl.ds(o, _L)]
                        cm[pl.ds(k * _L, _L)] = cm[pl.ds(o, _L)]
                        vm[pl.ds(k * _L, _L)] = vm[pl.ds(o, _L)]
                return rem

            cnt = lax.fori_loop(0, n_win, win, jnp.int32(0))

            for k in range(_RB // _L):
                vm[pl.ds(cnt + k * _L, _L)] = jnp.zeros((_L,), jnp.float32)
                cm[pl.ds(cnt + k * _L, _L)] = jnp.zeros((_L,), jnp.int32)

            @pl.when(cnt > 0)
            def _():
                flush(src, jnp.int32(0))

        def run_chunk(ph, base):
            ch = ph['CH']
            nblk = ch // _ZR
            out = outs[ph['out']]
            nb_t = (nblk + _NT - 1) // _NT
            for i in range(nb_t):
                b = sid + i * _NT
                @pl.when(b < nblk)
                def _(b=b):
                    pltpu.sync_copy(zrows, spmem.at[pl.ds(b * _ZR, _ZR)])
            plsc.subcore_barrier()
            for (t_ix, s_ix) in ph['tasks']:
                run_task(srcs[s_ix], tr[3 * t_ix], tr[3 * t_ix + 1],
                         tr[3 * t_ix + 2], base, ch)
            plsc.subcore_barrier()
            for i in range(nb_t):
                b = sid + i * _NT
                @pl.when(b < nblk)
                def _(b=b):
                    pltpu.sync_copy(spmem.at[pl.ds(b * _ZR, _ZR)],
                                    out.at[pl.ds(base + b * _ZR, _ZR)])

        for ph in phases:
            if ph['slots'] == 1:
                run_chunk(ph, cid * ph['CH'])
            else:
                def sl(s, _, ph=ph):
                    run_chunk(ph, (s * 2 + cid) * ph['CH'])
                    return 0
                lax.fori_loop(0, ph['slots'], sl, 0)

    return body


def _sc_call(phases, srcs, tasks, out_rows, spmem_rows):
    body = _build_sc_body(phases, len(srcs), len(tasks))
    scratch = [
        pltpu.VMEM_SHARED((spmem_rows, _C), jnp.float32),
        pltpu.VMEM((_W,), jnp.int32),
        pltpu.VMEM((_W,), jnp.int32),
        pltpu.VMEM((_W,), jnp.float32),
        pltpu.VMEM((_BUF,), jnp.int32),
        pltpu.VMEM((_BUF,), jnp.int32),
        pltpu.VMEM((_BUF,), jnp.float32),
        pltpu.VMEM((_RB,), jnp.int32),
        pltpu.VMEM((_RB,), jnp.int32),
        pltpu.VMEM((_RB, _C), jnp.float32),
        pltpu.VMEM((_ZR, _C), jnp.float32),
        pltpu.SemaphoreType.DMA,
    ]
    mesh = plsc.VectorSubcoreMesh(core_axis_name="c", subcore_axis_name="s")
    out_type = tuple(jax.ShapeDtypeStruct((n, _C), jnp.float32)
                     for n in out_rows)
    f = pl.kernel(body, out_type=out_type, mesh=mesh, scratch_types=scratch)
    flat = [a for t in tasks for a in t]
    return f(*srcs, *flat)


def _mm(x, ws, sig, br):
    n = x.shape[0]
    kdim = _C * len(ws)
    w = jnp.concatenate(ws, axis=1)

    def mk(x_ref, w_ref, *o_refs):
        xb = x_ref[...]
        if sig:
            xb = jax.nn.sigmoid(xb)
        res = jnp.dot(xb, w_ref[...], preferred_element_type=jnp.float32)
        for i, o in enumerate(o_refs):
            o[...] = res[:, i * _C:(i + 1) * _C]

    outs = pl.pallas_call(
        mk,
        grid=(n // br,),
        in_specs=[pl.BlockSpec((br, _C), lambda i: (i, 0)),
                  pl.BlockSpec((_C, kdim), lambda i: (0, 0))],
        out_specs=[pl.BlockSpec((br, _C), lambda i: (i, 0))] * len(ws),
        out_shape=[jax.ShapeDtypeStruct((n, _C), jnp.float32)] * len(ws),
    )(x, w)
    return outs if len(ws) > 1 else outs[0]


def _final_linear(x, w_lin, b, br):
    n = x.shape[0]

    def mk(x_ref, w_ref, b_ref, o_ref):
        xb = jax.nn.sigmoid(x_ref[...])
        o_ref[...] = (jnp.dot(xb, w_ref[...],
                              preferred_element_type=jnp.float32)
                      + b_ref[0, 0])

    out = pl.pallas_call(
        mk,
        grid=(n // br,),
        in_specs=[pl.BlockSpec((br, _C), lambda i: (i, 0)),
                  pl.BlockSpec((_C, 1), lambda i: (0, 0)),
                  pl.BlockSpec((1, 1), lambda i: (0, 0))],
        out_specs=pl.BlockSpec((br, 1), lambda i: (i, 0)),
        out_shape=jax.ShapeDtypeStruct((n, 1), jnp.float32),
    )(x, w_lin, b.reshape(1, 1))
    return out.reshape(n)


def kernel(x0, x1, x2, b1_row, b1_col, b1_val, b2_row, b2_col, b2_val,
           a0_row, a0_col, a0_val, a1_row, a1_col, a1_val, a2_row, a2_col,
           a2_val, params):
    p = params
    N0, N1 = x0.shape[0], x1.shape[0]

    y0s, y0l = _mm(x0, [p['W_same_0_0'], p['W_lth_0_1']], sig=False, br=2000)
    y1s, y1h = _mm(x1, [p['W_same_0_1'], p['W_htl_0_0']], sig=False, br=2000)
    y2h = _mm(x2, [p['W_htl_0_1']], sig=False, br=2000)

    t0 = _pad_task(a0_row, a0_col, a0_val, N0)
    t1 = _pad_task(b1_row, b1_col, b1_val, N0)
    t2 = _pad_task(a1_row, a1_col, a1_val, N1)
    t3 = _pad_task(b2_row, b2_col, b2_val, N1)
    t4 = _pad_task(b1_col, b1_row, b1_val, N1)

    phases0 = [
        dict(CH=5000, slots=1, out=0, tasks=[(0, 0), (1, 1)]),
        dict(CH=10000, slots=8, out=1, tasks=[(2, 2), (3, 3), (4, 4)]),
    ]
    m0, m1 = _sc_call(phases0, [y0s, y1h, y1s, y2h, y0l],
                      [t0, t1, t2, t3, t4], [N0, N1], 10000)

    z0 = _mm(m0, [p['W_same_1_0']], sig=True, br=2000)
    z1 = _mm(m1, [p['W_htl_1_0']], sig=True, br=2000)

    phases1 = [dict(CH=5000, slots=1, out=0, tasks=[(0, 0), (1, 1)])]
    (m0b,) = _sc_call(phases1, [z0, z1], [t0, t1], [N0], 5000)

    return _final_linear(m0b, p['W_lin'], p['b_lin'], br=2000)


# speedup vs baseline: 2.1588x; 1.5198x over previous
import jax
import jax.numpy as jnp
from jax import lax
from jax.experimental import pallas as pl
from jax.experimental.pallas import tpu as pltpu
from jax.experimental.pallas import tpu_sc as plsc

_L = 16
_NT = 16
_W = 2048
_GRP = _W // _L
_RB = 128
_BUF = 2 * _W + _RB + _L
_ZR = 40
_C = 128


def _pad_task(r, c, v, n_dest):
    n = r.shape[0]
    q = _NT * _W
    m = ((n + q - 1) // q) * q - n
    if m:
        r = jnp.concatenate([r.astype(jnp.int32),
                             jnp.full((m,), n_dest, jnp.int32)])
        c = jnp.concatenate([c.astype(jnp.int32), jnp.zeros((m,), jnp.int32)])
        v = jnp.concatenate([v, jnp.zeros((m,), v.dtype)])
    else:
        r = r.astype(jnp.int32)
        c = c.astype(jnp.int32)
    return r, c, v


def _gat16(x, idx):
    dn = lax.GatherDimensionNumbers(offset_dims=(), collapsed_slice_dims=(0,),
                                    start_index_map=(0,))
    return lax.gather(x, idx[:, None], dn, slice_sizes=(1,),
                      mode=lax.GatherScatterMode.PROMISE_IN_BOUNDS)


def _build_sc_body(phases, n_srcs, n_tasks):
    def body(*refs):
        srcs = refs[:n_srcs]
        tr = refs[n_srcs:n_srcs + 3 * n_tasks]
        outs = refs[n_srcs + 3 * n_tasks:-17]
        (spmem, ra, ca, va, rb2, cb2, vb2, rlocm, cm, vm,
         rlocf, cf, rows, zrows, sema, semb, gsem) = refs[-17:]

        cid = lax.axis_index("c")
        sid = lax.axis_index("s")

        def zz(i, _):
            for k in range(8):
                zrows[i, pl.ds(k * _L, _L)] = jnp.zeros((_L,), jnp.float32)
            return 0
        lax.fori_loop(0, _ZR, zz, 0)

        def zm(i, _):
            rlocm[pl.ds(i * _L, _L)] = jnp.zeros((_L,), jnp.int32)
            cm[pl.ds(i * _L, _L)] = jnp.zeros((_L,), jnp.int32)
            vm[pl.ds(i * _L, _L)] = jnp.zeros((_L,), jnp.float32)
            return 0
        lax.fori_loop(0, _BUF // _L, zm, 0)

        def flush(src, off):
            for k in range(_RB // _L):
                rlocf[pl.ds(k * _L, _L)] = rlocm[pl.ds(off + k * _L, _L)]
                cf[pl.ds(k * _L, _L)] = cm[pl.ds(off + k * _L, _L)]
            pltpu.async_copy(src.at[cf], rows, gsem).wait()

            def srow(i, _):
                vgrp = vm[pl.ds(off + i * _L, _L)]
                for jj in range(_L):
                    vj = _gat16(vgrp, jnp.full((_L,), jj, jnp.int32))
                    jr = i * _L + jj
                    for kk in range(_C // _L):
                        rows[jr, pl.ds(kk * _L, _L)] = (
                            rows[jr, pl.ds(kk * _L, _L)] * vj)
                return 0
            lax.fori_loop(0, _RB // _L, srow, 0)
            pltpu.sync_copy(rows, spmem.at[rlocf], add=True)

        def scan_window(bufs, base, ch, cnt):
            rbuf, cbuf, vbuf = bufs
            iota = lax.iota(jnp.int32, _L)

            def grp4(q, cnt):
                packed = []
                for u in range(4):
                    g = q * 4 + u
                    rv = rbuf[pl.ds(g * _L, _L)]
                    cv = cbuf[pl.ds(g * _L, _L)]
                    vv = vbuf[pl.ds(g * _L, _L)]
                    m = (rv >= base) & (rv < base + ch)
                    p = jnp.where(m, 1, 0)
                    for s in (1, 2, 4, 8):
                        w = _gat16(p, jnp.maximum(iota - s, 0))
                        p = p + jnp.where(iota >= s, w, 0)
                    total = p[15]
                    tgt = iota + 1
                    lo = jnp.zeros((_L,), jnp.int32)
                    hi = jnp.full((_L,), _L - 1, jnp.int32)
                    for _ in range(4):
                        mid = lax.shift_right_logical(lo + hi, 1)
                        ge = _gat16(p, mid) >= tgt
                        hi = jnp.where(ge, mid, hi)
                        lo = jnp.where(ge, lo, mid + 1)
                    rloc = jnp.clip(_gat16(rv, hi) - base, 0, ch - 1)
                    packed.append((rloc, _gat16(cv, hi), _gat16(vv, hi),
                                   total))
                for rloc, cc, vc, total in packed:
                    rlocm[pl.ds(cnt, _L)] = rloc
                    cm[pl.ds(cnt, _L)] = cc
                    vm[pl.ds(cnt, _L)] = vc
                    cnt = cnt + total
                return cnt
            return lax.fori_loop(0, _GRP // 4, grp4, cnt)

        def run_task(src, r_h, c_h, v_h, base, ch):
            per_tile = r_h.shape[0] // _NT
            n_win = per_tile // _W

            def issue(w, bufs, sem):
                start = sid * per_tile + w * _W
                pltpu.async_copy(r_h.at[pl.ds(start, _W)], bufs[0], sem)
                pltpu.async_copy(c_h.at[pl.ds(start, _W)], bufs[1], sem)
                pltpu.async_copy(v_h.at[pl.ds(start, _W)], bufs[2], sem)

            def drain(w, bufs, sem):
                start = sid * per_tile + w * _W
                pltpu.make_async_copy(r_h.at[pl.ds(start, _W)], bufs[0],
                                      sem).wait()
                pltpu.make_async_copy(c_h.at[pl.ds(start, _W)], bufs[1],
                                      sem).wait()
                pltpu.make_async_copy(v_h.at[pl.ds(start, _W)], bufs[2],
                                      sem).wait()

            bufs_a = (ra, ca, va)
            bufs_b = (rb2, cb2, vb2)
            issue(0, bufs_a, sema)

            def pair(k, cnt):
                wa = 2 * k
                issue(wa + 1, bufs_b, semb)
                drain(wa, bufs_a, sema)
                cnt = scan_window(bufs_a, base, ch, cnt)

                @pl.when(wa + 2 < n_win)
                def _():
                    issue(wa + 2, bufs_a, sema)
                drain(wa + 1, bufs_b, semb)
                cnt = scan_window(bufs_b, base, ch, cnt)

                nfull = cnt // _RB

                def fb(i, _):
                    flush(src, i * _RB)
                    return 0
                lax.fori_loop(0, nfull, fb, 0)

                rem = cnt - nfull * _RB
                for k2 in range(_RB // _L):
                    @pl.when(k2 * _L < rem)
                    def _(k2=k2):
                        o = nfull * _RB + k2 * _L
                        rlocm[pl.ds(k2 * _L, _L)] = rlocm[pl.ds(o, _L)]
                        cm[pl.ds(k2 * _L, _L)] = cm[pl.ds(o, _L)]
                        vm[pl.ds(k2 * _L, _L)] = vm[pl.ds(o, _L)]
                return rem

            cnt = lax.fori_loop(0, n_win // 2, pair, jnp.int32(0))

            for k in range(_RB // _L):
                vm[pl.ds(cnt + k * _L, _L)] = jnp.zeros((_L,), jnp.float32)
                cm[pl.ds(cnt + k * _L, _L)] = jnp.zeros((_L,), jnp.int32)

            @pl.when(cnt > 0)
            def _():
                flush(src, jnp.int32(0))

        def run_chunk(ph, base):
            ch = ph['CH']
            nblk = ch // _ZR
            out = outs[ph['out']]
            nb_t = (nblk + _NT - 1) // _NT
            for i in range(nb_t):
                b = sid + i * _NT
                @pl.when(b < nblk)
                def _(b=b):
                    pltpu.sync_copy(zrows, spmem.at[pl.ds(b * _ZR, _ZR)])
            plsc.subcore_barrier()
            for (t_ix, s_ix) in ph['tasks']:
                run_task(srcs[s_ix], tr[3 * t_ix], tr[3 * t_ix + 1],
                         tr[3 * t_ix + 2], base, ch)
            plsc.subcore_barrier()
            for i in range(nb_t):
                b = sid + i * _NT
                @pl.when(b < nblk)
                def _(b=b):
                    pltpu.sync_copy(spmem.at[pl.ds(b * _ZR, _ZR)],
                                    out.at[pl.ds(base + b * _ZR, _ZR)])

        for ph in phases:
            if ph['slots'] == 1:
                run_chunk(ph, cid * ph['CH'])
            else:
                def sl(s, _, ph=ph):
                    run_chunk(ph, (s * 2 + cid) * ph['CH'])
                    return 0
                lax.fori_loop(0, ph['slots'], sl, 0)

    return body


def _sc_call(phases, srcs, tasks, out_rows, spmem_rows):
    body = _build_sc_body(phases, len(srcs), len(tasks))
    scratch = [
        pltpu.VMEM_SHARED((spmem_rows, _C), jnp.float32),
        pltpu.VMEM((_W,), jnp.int32),
        pltpu.VMEM((_W,), jnp.int32),
        pltpu.VMEM((_W,), jnp.float32),
        pltpu.VMEM((_W,), jnp.int32),
        pltpu.VMEM((_W,), jnp.int32),
        pltpu.VMEM((_W,), jnp.float32),
        pltpu.VMEM((_BUF,), jnp.int32),
        pltpu.VMEM((_BUF,), jnp.int32),
        pltpu.VMEM((_BUF,), jnp.float32),
        pltpu.VMEM((_RB,), jnp.int32),
        pltpu.VMEM((_RB,), jnp.int32),
        pltpu.VMEM((_RB, _C), jnp.float32),
        pltpu.VMEM((_ZR, _C), jnp.float32),
        pltpu.SemaphoreType.DMA,
        pltpu.SemaphoreType.DMA,
        pltpu.SemaphoreType.DMA,
    ]
    mesh = plsc.VectorSubcoreMesh(core_axis_name="c", subcore_axis_name="s")
    out_type = tuple(jax.ShapeDtypeStruct((n, _C), jnp.float32)
                     for n in out_rows)
    f = pl.kernel(body, out_type=out_type, mesh=mesh, scratch_types=scratch)
    flat = [a for t in tasks for a in t]
    return f(*srcs, *flat)


def _mm(x, ws, sig, br):
    n = x.shape[0]
    kdim = _C * len(ws)
    w = jnp.concatenate(ws, axis=1)

    def mk(x_ref, w_ref, *o_refs):
        xb = x_ref[...]
        if sig:
            xb = jax.nn.sigmoid(xb)
        res = jnp.dot(xb, w_ref[...], preferred_element_type=jnp.float32)
        for i, o in enumerate(o_refs):
            o[...] = res[:, i * _C:(i + 1) * _C]

    outs = pl.pallas_call(
        mk,
        grid=(n // br,),
        in_specs=[pl.BlockSpec((br, _C), lambda i: (i, 0)),
                  pl.BlockSpec((_C, kdim), lambda i: (0, 0))],
        out_specs=[pl.BlockSpec((br, _C), lambda i: (i, 0))] * len(ws),
        out_shape=[jax.ShapeDtypeStruct((n, _C), jnp.float32)] * len(ws),
    )(x, w)
    return outs if len(ws) > 1 else outs[0]


def _final_linear(x, w_lin, b, br):
    n = x.shape[0]

    def mk(x_ref, w_ref, b_ref, o_ref):
        xb = jax.nn.sigmoid(x_ref[...])
        o_ref[...] = (jnp.dot(xb, w_ref[...],
                              preferred_element_type=jnp.float32)
                      + b_ref[0, 0])

    out = pl.pallas_call(
        mk,
        grid=(n // br,),
        in_specs=[pl.BlockSpec((br, _C), lambda i: (i, 0)),
                  pl.BlockSpec((_C, 1), lambda i: (0, 0)),
                  pl.BlockSpec((1, 1), lambda i: (0, 0))],
        out_specs=pl.BlockSpec((br, 1), lambda i: (i, 0)),
        out_shape=jax.ShapeDtypeStruct((n, 1), jnp.float32),
    )(x, w_lin, b.reshape(1, 1))
    return out.reshape(n)


def kernel(x0, x1, x2, b1_row, b1_col, b1_val, b2_row, b2_col, b2_val,
           a0_row, a0_col, a0_val, a1_row, a1_col, a1_val, a2_row, a2_col,
           a2_val, params):
    p = params
    N0, N1 = x0.shape[0], x1.shape[0]

    y0s, y0l = _mm(x0, [p['W_same_0_0'], p['W_lth_0_1']], sig=False, br=2000)
    y1s, y1h = _mm(x1, [p['W_same_0_1'], p['W_htl_0_0']], sig=False, br=2000)
    y2h = _mm(x2, [p['W_htl_0_1']], sig=False, br=2000)

    t0 = _pad_task(a0_row, a0_col, a0_val, N0)
    t1 = _pad_task(b1_row, b1_col, b1_val, N0)
    t2 = _pad_task(a1_row, a1_col, a1_val, N1)
    t3 = _pad_task(b2_row, b2_col, b2_val, N1)
    t4 = _pad_task(b1_col, b1_row, b1_val, N1)

    phases0 = [
        dict(CH=5000, slots=1, out=0, tasks=[(0, 0), (1, 1)]),
        dict(CH=10000, slots=8, out=1, tasks=[(2, 2), (3, 3), (4, 4)]),
    ]
    m0, m1 = _sc_call(phases0, [y0s, y1h, y1s, y2h, y0l],
                      [t0, t1, t2, t3, t4], [N0, N1], 10000)

    z0 = _mm(m0, [p['W_same_1_0']], sig=True, br=2000)
    z1 = _mm(m1, [p['W_htl_1_0']], sig=True, br=2000)

    phases1 = [dict(CH=5000, slots=1, out=0, tasks=[(0, 0), (1, 1)])]
    (m0b,) = _sc_call(phases1, [z0, z1], [t0, t1], [N0], 5000)

    return _final_linear(m0b, p['W_lin'], p['b_lin'], br=2000)


# speedup vs baseline: 2.1646x; 1.0027x over previous
import jax
import jax.numpy as jnp
from jax import lax
from jax.experimental import pallas as pl
from jax.experimental.pallas import tpu as pltpu
from jax.experimental.pallas import tpu_sc as plsc

_L = 16
_NT = 16
_W = 2048
_GRP = _W // _L
_RB = 128
_BUF = 2 * _W + _RB + _L
_ZR = 40
_C = 128


def _pad_task(r, c, v, n_dest):
    n = r.shape[0]
    q = _NT * _W
    m = ((n + q - 1) // q) * q - n
    if m:
        r = jnp.concatenate([r.astype(jnp.int32),
                             jnp.full((m,), n_dest, jnp.int32)])
        c = jnp.concatenate([c.astype(jnp.int32), jnp.zeros((m,), jnp.int32)])
        v = jnp.concatenate([v, jnp.zeros((m,), v.dtype)])
    else:
        r = r.astype(jnp.int32)
        c = c.astype(jnp.int32)
    return r, c, v


def _gat16(x, idx):
    dn = lax.GatherDimensionNumbers(offset_dims=(), collapsed_slice_dims=(0,),
                                    start_index_map=(0,))
    return lax.gather(x, idx[:, None], dn, slice_sizes=(1,),
                      mode=lax.GatherScatterMode.PROMISE_IN_BOUNDS)


def _build_sc_body(phases, n_srcs, n_tasks):
    def body(*refs):
        srcs = refs[:n_srcs]
        tr = refs[n_srcs:n_srcs + 3 * n_tasks]
        outs = refs[n_srcs + 3 * n_tasks:-17]
        (spmem, ra, ca, va, rb2, cb2, vb2, rlocm, cm, vm,
         rlocf, cf, rows, zrows, sema, semb, gsem) = refs[-17:]

        cid = lax.axis_index("c")
        sid = lax.axis_index("s")

        def zz(i, _):
            for k in range(8):
                zrows[i, pl.ds(k * _L, _L)] = jnp.zeros((_L,), jnp.float32)
            return 0
        lax.fori_loop(0, _ZR, zz, 0)

        def zm(i, _):
            rlocm[pl.ds(i * _L, _L)] = jnp.zeros((_L,), jnp.int32)
            cm[pl.ds(i * _L, _L)] = jnp.zeros((_L,), jnp.int32)
            vm[pl.ds(i * _L, _L)] = jnp.zeros((_L,), jnp.float32)
            return 0
        lax.fori_loop(0, _BUF // _L, zm, 0)

        def flush(src, off):
            for k in range(_RB // _L):
                rlocf[pl.ds(k * _L, _L)] = rlocm[pl.ds(off + k * _L, _L)]
                cf[pl.ds(k * _L, _L)] = cm[pl.ds(off + k * _L, _L)]
            pltpu.async_copy(src.at[cf], rows, gsem).wait()

            def srow(i, _):
                vgrp = vm[pl.ds(off + i * _L, _L)]
                for jj in range(_L):
                    vj = _gat16(vgrp, jnp.full((_L,), jj, jnp.int32))
                    jr = i * _L + jj
                    for kk in range(_C // _L):
                        rows[jr, pl.ds(kk * _L, _L)] = (
                            rows[jr, pl.ds(kk * _L, _L)] * vj)
                return 0
            lax.fori_loop(0, _RB // _L, srow, 0)
            pltpu.sync_copy(rows, spmem.at[rlocf], add=True)

        def scan_window(bufs, base, ch, cnt):
            rbuf, cbuf, vbuf = bufs
            iota = lax.iota(jnp.int32, _L)

            def grp4(q, cnt):
                packed = []
                for u in range(4):
                    g = q * 4 + u
                    rv = rbuf[pl.ds(g * _L, _L)]
                    cv = cbuf[pl.ds(g * _L, _L)]
                    vv = vbuf[pl.ds(g * _L, _L)]
                    m = (rv >= base) & (rv < base + ch)
                    p = jnp.where(m, 1, 0)
                    for s in (1, 2, 4, 8):
                        w = _gat16(p, jnp.maximum(iota - s, 0))
                        p = p + jnp.where(iota >= s, w, 0)
                    tgt = iota + 1
                    lo = jnp.zeros((_L,), jnp.int32)
                    hi = jnp.full((_L,), _L - 1, jnp.int32)
                    for _ in range(4):
                        mid = lax.shift_right_logical(lo + hi, 1)
                        ge = _gat16(p, mid) >= tgt
                        hi = jnp.where(ge, mid, hi)
                        lo = jnp.where(ge, lo, mid + 1)
                    rloc = jnp.clip(_gat16(rv, hi) - base, 0, ch - 1)
                    packed.append((rloc, _gat16(cv, hi), _gat16(vv, hi),
                                   p[15]))
                off = 0
                for rloc, cc, vc, total in packed:
                    at = cnt + off
                    rlocm[pl.ds(at, _L)] = rloc
                    cm[pl.ds(at, _L)] = cc
                    vm[pl.ds(at, _L)] = vc
                    off = off + total
                return cnt + off
            return lax.fori_loop(0, _GRP // 4, grp4, cnt)

        def run_task(src, r_h, c_h, v_h, base, ch):
            per_tile = r_h.shape[0] // _NT
            n_win = per_tile // _W

            def issue(w, bufs, sem):
                start = sid * per_tile + w * _W
                pltpu.async_copy(r_h.at[pl.ds(start, _W)], bufs[0], sem)
                pltpu.async_copy(c_h.at[pl.ds(start, _W)], bufs[1], sem)
                pltpu.async_copy(v_h.at[pl.ds(start, _W)], bufs[2], sem)

            def drain(w, bufs, sem):
                start = sid * per_tile + w * _W
                pltpu.make_async_copy(r_h.at[pl.ds(start, _W)], bufs[0],
                                      sem).wait()
                pltpu.make_async_copy(c_h.at[pl.ds(start, _W)], bufs[1],
                                      sem).wait()
                pltpu.make_async_copy(v_h.at[pl.ds(start, _W)], bufs[2],
                                      sem).wait()

            bufs_a = (ra, ca, va)
            bufs_b = (rb2, cb2, vb2)
            issue(0, bufs_a, sema)

            def pair(k, cnt):
                wa = 2 * k
                issue(wa + 1, bufs_b, semb)
                drain(wa, bufs_a, sema)
                cnt = scan_window(bufs_a, base, ch, cnt)

                @pl.when(wa + 2 < n_win)
                def _():
                    issue(wa + 2, bufs_a, sema)
                drain(wa + 1, bufs_b, semb)
                cnt = scan_window(bufs_b, base, ch, cnt)

                nfull = cnt // _RB

                def fb(i, _):
                    flush(src, i * _RB)
                    return 0
                lax.fori_loop(0, nfull, fb, 0)

                rem = cnt - nfull * _RB
                for k2 in range(_RB // _L):
                    @pl.when(k2 * _L < rem)
                    def _(k2=k2):
                        o = nfull * _RB + k2 * _L
                        rlocm[pl.ds(k2 * _L, _L)] = rlocm[pl.ds(o, _L)]
                        cm[pl.ds(k2 * _L, _L)] = cm[pl.ds(o, _L)]
                        vm[pl.ds(k2 * _L, _L)] = vm[pl.ds(o, _L)]
                return rem

            cnt = lax.fori_loop(0, n_win // 2, pair, jnp.int32(0))

            for k in range(_RB // _L):
                vm[pl.ds(cnt + k * _L, _L)] = jnp.zeros((_L,), jnp.float32)
                cm[pl.ds(cnt + k * _L, _L)] = jnp.zeros((_L,), jnp.int32)

            @pl.when(cnt > 0)
            def _():
                flush(src, jnp.int32(0))

        def run_chunk(ph, base):
            ch = ph['CH']
            nblk = ch // _ZR
            out = outs[ph['out']]
            nb_t = (nblk + _NT - 1) // _NT
            for i in range(nb_t):
                b = sid + i * _NT
                @pl.when(b < nblk)
                def _(b=b):
                    pltpu.sync_copy(zrows, spmem.at[pl.ds(b * _ZR, _ZR)])
            plsc.subcore_barrier()
            for (t_ix, s_ix) in ph['tasks']:
                run_task(srcs[s_ix], tr[3 * t_ix], tr[3 * t_ix + 1],
                         tr[3 * t_ix + 2], base, ch)
            plsc.subcore_barrier()
            for i in range(nb_t):
                b = sid + i * _NT
                @pl.when(b < nblk)
                def _(b=b):
                    pltpu.sync_copy(spmem.at[pl.ds(b * _ZR, _ZR)],
                                    out.at[pl.ds(base + b * _ZR, _ZR)])

        for ph in phases:
            if ph['slots'] == 1:
                run_chunk(ph, cid * ph['CH'])
            else:
                def sl(s, _, ph=ph):
                    run_chunk(ph, (s * 2 + cid) * ph['CH'])
                    return 0
                lax.fori_loop(0, ph['slots'], sl, 0)

    return body


def _sc_call(phases, srcs, tasks, out_rows, spmem_rows):
    body = _build_sc_body(phases, len(srcs), len(tasks))
    scratch = [
        pltpu.VMEM_SHARED((spmem_rows, _C), jnp.float32),
        pltpu.VMEM((_W,), jnp.int32),
        pltpu.VMEM((_W,), jnp.int32),
        pltpu.VMEM((_W,), jnp.float32),
        pltpu.VMEM((_W,), jnp.int32),
        pltpu.VMEM((_W,), jnp.int32),
        pltpu.VMEM((_W,), jnp.float32),
        pltpu.VMEM((_BUF,), jnp.int32),
        pltpu.VMEM((_BUF,), jnp.int32),
        pltpu.VMEM((_BUF,), jnp.float32),
        pltpu.VMEM((_RB,), jnp.int32),
        pltpu.VMEM((_RB,), jnp.int32),
        pltpu.VMEM((_RB, _C), jnp.float32),
        pltpu.VMEM((_ZR, _C), jnp.float32),
        pltpu.SemaphoreType.DMA,
        pltpu.SemaphoreType.DMA,
        pltpu.SemaphoreType.DMA,
    ]
    mesh = plsc.VectorSubcoreMesh(core_axis_name="c", subcore_axis_name="s")
    out_type = tuple(jax.ShapeDtypeStruct((n, _C), jnp.float32)
                     for n in out_rows)
    f = pl.kernel(body, out_type=out_type, mesh=mesh, scratch_types=scratch)
    flat = [a for t in tasks for a in t]
    return f(*srcs, *flat)


def _mm(x, ws, sig, br):
    n = x.shape[0]
    kdim = _C * len(ws)
    w = jnp.concatenate(ws, axis=1)

    def mk(x_ref, w_ref, *o_refs):
        xb = x_ref[...]
        if sig:
            xb = jax.nn.sigmoid(xb)
        res = jnp.dot(xb, w_ref[...], preferred_element_type=jnp.float32)
        for i, o in enumerate(o_refs):
            o[...] = res[:, i * _C:(i + 1) * _C]

    outs = pl.pallas_call(
        mk,
        grid=(n // br,),
        in_specs=[pl.BlockSpec((br, _C), lambda i: (i, 0)),
                  pl.BlockSpec((_C, kdim), lambda i: (0, 0))],
        out_specs=[pl.BlockSpec((br, _C), lambda i: (i, 0))] * len(ws),
        out_shape=[jax.ShapeDtypeStruct((n, _C), jnp.float32)] * len(ws),
    )(x, w)
    return outs if len(ws) > 1 else outs[0]


def _final_linear(x, w_lin, b, br):
    n = x.shape[0]

    def mk(x_ref, w_ref, b_ref, o_ref):
        xb = jax.nn.sigmoid(x_ref[...])
        o_ref[...] = (jnp.dot(xb, w_ref[...],
                              preferred_element_type=jnp.float32)
                      + b_ref[0, 0])

    out = pl.pallas_call(
        mk,
        grid=(n // br,),
        in_specs=[pl.BlockSpec((br, _C), lambda i: (i, 0)),
                  pl.BlockSpec((_C, 1), lambda i: (0, 0)),
                  pl.BlockSpec((1, 1), lambda i: (0, 0))],
        out_specs=pl.BlockSpec((br, 1), lambda i: (i, 0)),
        out_shape=jax.ShapeDtypeStruct((n, 1), jnp.float32),
    )(x, w_lin, b.reshape(1, 1))
    return out.reshape(n)


def kernel(x0, x1, x2, b1_row, b1_col, b1_val, b2_row, b2_col, b2_val,
           a0_row, a0_col, a0_val, a1_row, a1_col, a1_val, a2_row, a2_col,
           a2_val, params):
    p = params
    N0, N1 = x0.shape[0], x1.shape[0]

    y0s, y0l = _mm(x0, [p['W_same_0_0'], p['W_lth_0_1']], sig=False, br=2000)
    y1s, y1h = _mm(x1, [p['W_same_0_1'], p['W_htl_0_0']], sig=False, br=2000)
    y2h = _mm(x2, [p['W_htl_0_1']], sig=False, br=2000)

    t0 = _pad_task(a0_row, a0_col, a0_val, N0)
    t1 = _pad_task(b1_row, b1_col, b1_val, N0)
    t2 = _pad_task(a1_row, a1_col, a1_val, N1)
    t3 = _pad_task(b2_row, b2_col, b2_val, N1)
    t4 = _pad_task(b1_col, b1_row, b1_val, N1)

    phases0 = [
        dict(CH=5000, slots=1, out=0, tasks=[(0, 0), (1, 1)]),
        dict(CH=10000, slots=8, out=1, tasks=[(2, 2), (3, 3), (4, 4)]),
    ]
    m0, m1 = _sc_call(phases0, [y0s, y1h, y1s, y2h, y0l],
                      [t0, t1, t2, t3, t4], [N0, N1], 10000)

    z0 = _mm(m0, [p['W_same_1_0']], sig=True, br=2000)
    z1 = _mm(m1, [p['W_htl_1_0']], sig=True, br=2000)

    phases1 = [dict(CH=5000, slots=1, out=0, tasks=[(0, 0), (1, 1)])]
    (m0b,) = _sc_call(phases1, [z0, z1], [t0, t1], [N0], 5000)

    return _final_linear(m0b, p['W_lin'], p['b_lin'], br=2000)


# speedup vs baseline: 2.2032x; 1.0178x over previous
import jax
import jax.numpy as jnp
from jax import lax
from jax.experimental import pallas as pl
from jax.experimental.pallas import tpu as pltpu
from jax.experimental.pallas import tpu_sc as plsc

_L = 16
_NT = 16
_W = 2048
_GRP = _W // _L
_RB = 64
_BUF = 2 * _W + _RB + _L
_ZR = 40
_C = 128


def _pad_task(r, c, v, n_dest):
    n = r.shape[0]
    q = _NT * _W
    m = ((n + q - 1) // q) * q - n
    if m:
        r = jnp.concatenate([r.astype(jnp.int32),
                             jnp.full((m,), n_dest, jnp.int32)])
        c = jnp.concatenate([c.astype(jnp.int32), jnp.zeros((m,), jnp.int32)])
        v = jnp.concatenate([v, jnp.zeros((m,), v.dtype)])
    else:
        r = r.astype(jnp.int32)
        c = c.astype(jnp.int32)
    return r, c, v


def _gat16(x, idx):
    dn = lax.GatherDimensionNumbers(offset_dims=(), collapsed_slice_dims=(0,),
                                    start_index_map=(0,))
    return lax.gather(x, idx[:, None], dn, slice_sizes=(1,),
                      mode=lax.GatherScatterMode.PROMISE_IN_BOUNDS)


def _build_sc_body(phases, n_srcs, n_tasks):
    def body(*refs):
        srcs = refs[:n_srcs]
        tr = refs[n_srcs:n_srcs + 3 * n_tasks]
        outs = refs[n_srcs + 3 * n_tasks:-23]
        (spmem, ra, ca, va, rb2, cb2, vb2, rlocm, cm, vm,
         rlocf0, cf0, rows0, rlocf1, cf1, rows1, zrows,
         sema, semb, gsem0, gsem1, ssem0, ssem1) = refs[-23:]
        rset = ((rlocf0, cf0, rows0, gsem0, ssem0),
                (rlocf1, cf1, rows1, gsem1, ssem1))

        cid = lax.axis_index("c")
        sid = lax.axis_index("s")

        def zz(i, _):
            for k in range(8):
                zrows[i, pl.ds(k * _L, _L)] = jnp.zeros((_L,), jnp.float32)
            return 0
        lax.fori_loop(0, _ZR, zz, 0)

        def zm(i, _):
            rlocm[pl.ds(i * _L, _L)] = jnp.zeros((_L,), jnp.int32)
            cm[pl.ds(i * _L, _L)] = jnp.zeros((_L,), jnp.int32)
            vm[pl.ds(i * _L, _L)] = jnp.zeros((_L,), jnp.float32)
            return 0
        lax.fori_loop(0, _BUF // _L, zm, 0)

        def _scale(rows, off):
            def srow(i, _):
                vgrp = vm[pl.ds(off + i * 8, _L)]
                for jj in range(8):
                    vj = _gat16(vgrp, jnp.full((_L,), jj, jnp.int32))
                    jr = i * 8 + jj
                    for kk in range(_C // _L):
                        rows[jr, pl.ds(kk * _L, _L)] = (
                            rows[jr, pl.ds(kk * _L, _L)] * vj)
                return 0
            lax.fori_loop(0, _RB // 8, srow, 0)

        def _stage(src, s, off):
            rlocf, cf, rows, gsem, _ = rset[s]
            for k in range(_RB // _L):
                rlocf[pl.ds(k * _L, _L)] = rlocm[pl.ds(off + k * _L, _L)]
                cf[pl.ds(k * _L, _L)] = cm[pl.ds(off + k * _L, _L)]
            pltpu.async_copy(src.at[cf], rows, gsem)

        def _wait_scatter(s):
            rlocf, _, rows, _, ssem = rset[s]
            pltpu.make_async_copy(rows, spmem.at[rlocf], ssem).wait()

        def _finish(src, s, off):
            rlocf, cf, rows, gsem, ssem = rset[s]
            pltpu.make_async_copy(src.at[cf], rows, gsem).wait()
            _scale(rows, off)
            pltpu.async_copy(rows, spmem.at[rlocf], ssem, add=True)

        def flush_all(src, cnt):
            nfull = cnt // _RB

            @pl.when(nfull > 0)
            def _():
                _stage(src, 0, jnp.int32(0))

            def fb(i, _):
                for s in (0, 1):
                    @pl.when((i % 2) == s)
                    def _(s=s):
                        o = (i + 1) * _RB

                        @pl.when(i + 1 < nfull)
                        def _():
                            @pl.when(i >= 1)
                            def _():
                                _wait_scatter(1 - s)
                            _stage(src, 1 - s, o)
                        _finish(src, s, i * _RB)
                return 0
            lax.fori_loop(0, nfull, fb, 0)

            for s in (0, 1):
                @pl.when(((nfull - 1) % 2 == s) & (nfull >= 1))
                def _(s=s):
                    _wait_scatter(s)

                @pl.when((nfull % 2 == s) & (nfull >= 2))
                def _(s=s):
                    _wait_scatter(s)
            return nfull

        def scan_window(bufs, base, ch, cnt):
            rbuf, cbuf, vbuf = bufs
            iota = lax.iota(jnp.int32, _L)

            def grp4(q, cnt):
                packed = []
                for u in range(4):
                    g = q * 4 + u
                    rv = rbuf[pl.ds(g * _L, _L)]
                    cv = cbuf[pl.ds(g * _L, _L)]
                    vv = vbuf[pl.ds(g * _L, _L)]
                    m = (rv >= base) & (rv < base + ch)
                    p = jnp.where(m, 1, 0)
                    for s in (1, 2, 4, 8):
                        w = _gat16(p, jnp.maximum(iota - s, 0))
                        p = p + jnp.where(iota >= s, w, 0)
                    tgt = iota + 1
                    lo = jnp.zeros((_L,), jnp.int32)
                    hi = jnp.full((_L,), _L - 1, jnp.int32)
                    for _ in range(4):
                        mid = lax.shift_right_logical(lo + hi, 1)
                        ge = _gat16(p, mid) >= tgt
                        hi = jnp.where(ge, mid, hi)
                        lo = jnp.where(ge, lo, mid + 1)
                    rloc = jnp.clip(_gat16(rv, hi) - base, 0, ch - 1)
                    packed.append((rloc, _gat16(cv, hi), _gat16(vv, hi),
                                   p[15]))
                off = 0
                for rloc, cc, vc, total in packed:
                    at = cnt + off
                    rlocm[pl.ds(at, _L)] = rloc
                    cm[pl.ds(at, _L)] = cc
                    vm[pl.ds(at, _L)] = vc
                    off = off + total
                return cnt + off
            return lax.fori_loop(0, _GRP // 4, grp4, cnt)

        def run_task(src, r_h, c_h, v_h, base, ch):
            per_tile = r_h.shape[0] // _NT
            n_win = per_tile // _W

            def issue(w, bufs, sem):
                start = sid * per_tile + w * _W
                pltpu.async_copy(r_h.at[pl.ds(start, _W)], bufs[0], sem)
                pltpu.async_copy(c_h.at[pl.ds(start, _W)], bufs[1], sem)
                pltpu.async_copy(v_h.at[pl.ds(start, _W)], bufs[2], sem)

            def drain(w, bufs, sem):
                start = sid * per_tile + w * _W
                pltpu.make_async_copy(r_h.at[pl.ds(start, _W)], bufs[0],
                                      sem).wait()
                pltpu.make_async_copy(c_h.at[pl.ds(start, _W)], bufs[1],
                                      sem).wait()
                pltpu.make_async_copy(v_h.at[pl.ds(start, _W)], bufs[2],
                                      sem).wait()

            bufs_a = (ra, ca, va)
            bufs_b = (rb2, cb2, vb2)
            issue(0, bufs_a, sema)

            def pair(k, cnt):
                wa = 2 * k
                issue(wa + 1, bufs_b, semb)
                drain(wa, bufs_a, sema)
                cnt = scan_window(bufs_a, base, ch, cnt)

                @pl.when(wa + 2 < n_win)
                def _():
                    issue(wa + 2, bufs_a, sema)
                drain(wa + 1, bufs_b, semb)
                cnt = scan_window(bufs_b, base, ch, cnt)

                nfull = flush_all(src, cnt)

                rem = cnt - nfull * _RB
                for k2 in range(_RB // _L):
                    @pl.when(k2 * _L < rem)
                    def _(k2=k2):
                        o = nfull * _RB + k2 * _L
                        rlocm[pl.ds(k2 * _L, _L)] = rlocm[pl.ds(o, _L)]
                        cm[pl.ds(k2 * _L, _L)] = cm[pl.ds(o, _L)]
                        vm[pl.ds(k2 * _L, _L)] = vm[pl.ds(o, _L)]
                return rem

            cnt = lax.fori_loop(0, n_win // 2, pair, jnp.int32(0))

            for k in range(_RB // _L):
                vm[pl.ds(cnt + k * _L, _L)] = jnp.zeros((_L,), jnp.float32)
                cm[pl.ds(cnt + k * _L, _L)] = jnp.zeros((_L,), jnp.int32)

            @pl.when(cnt > 0)
            def _():
                _stage(src, 0, jnp.int32(0))
                rlocf, cf, rows, gsem, _ = rset[0]
                pltpu.make_async_copy(src.at[cf], rows, gsem).wait()
                _scale(rows, jnp.int32(0))
                pltpu.sync_copy(rows, spmem.at[rlocf], add=True)

        def run_chunk(ph, base):
            ch = ph['CH']
            nblk = ch // _ZR
            out = outs[ph['out']]
            nb_t = (nblk + _NT - 1) // _NT
            for i in range(nb_t):
                b = sid + i * _NT
                @pl.when(b < nblk)
                def _(b=b):
                    pltpu.sync_copy(zrows, spmem.at[pl.ds(b * _ZR, _ZR)])
            plsc.subcore_barrier()
            for (t_ix, s_ix) in ph['tasks']:
                run_task(srcs[s_ix], tr[3 * t_ix], tr[3 * t_ix + 1],
                         tr[3 * t_ix + 2], base, ch)
            plsc.subcore_barrier()
            for i in range(nb_t):
                b = sid + i * _NT
                @pl.when(b < nblk)
                def _(b=b):
                    pltpu.sync_copy(spmem.at[pl.ds(b * _ZR, _ZR)],
                                    out.at[pl.ds(base + b * _ZR, _ZR)])

        for ph in phases:
            if ph['slots'] == 1:
                run_chunk(ph, cid * ph['CH'])
            else:
                def sl(s, _, ph=ph):
                    run_chunk(ph, (s * 2 + cid) * ph['CH'])
                    return 0
                lax.fori_loop(0, ph['slots'], sl, 0)

    return body


def _sc_call(phases, srcs, tasks, out_rows, spmem_rows):
    body = _build_sc_body(phases, len(srcs), len(tasks))
    scratch = [
        pltpu.VMEM_SHARED((spmem_rows, _C), jnp.float32),
        pltpu.VMEM((_W,), jnp.int32),
        pltpu.VMEM((_W,), jnp.int32),
        pltpu.VMEM((_W,), jnp.float32),
        pltpu.VMEM((_W,), jnp.int32),
        pltpu.VMEM((_W,), jnp.int32),
        pltpu.VMEM((_W,), jnp.float32),
        pltpu.VMEM((_BUF,), jnp.int32),
        pltpu.VMEM((_BUF,), jnp.int32),
        pltpu.VMEM((_BUF,), jnp.float32),
        pltpu.VMEM((_RB,), jnp.int32),
        pltpu.VMEM((_RB,), jnp.int32),
        pltpu.VMEM((_RB, _C), jnp.float32),
        pltpu.VMEM((_RB,), jnp.int32),
        pltpu.VMEM((_RB,), jnp.int32),
        pltpu.VMEM((_RB, _C), jnp.float32),
        pltpu.VMEM((_ZR, _C), jnp.float32),
        pltpu.SemaphoreType.DMA,
        pltpu.SemaphoreType.DMA,
        pltpu.SemaphoreType.DMA,
        pltpu.SemaphoreType.DMA,
        pltpu.SemaphoreType.DMA,
        pltpu.SemaphoreType.DMA,
    ]
    mesh = plsc.VectorSubcoreMesh(core_axis_name="c", subcore_axis_name="s")
    out_type = tuple(jax.ShapeDtypeStruct((n, _C), jnp.float32)
                     for n in out_rows)
    f = pl.kernel(body, out_type=out_type, mesh=mesh, scratch_types=scratch)
    flat = [a for t in tasks for a in t]
    return f(*srcs, *flat)


def _mm(x, ws, sig, br):
    n = x.shape[0]
    kdim = _C * len(ws)
    w = jnp.concatenate(ws, axis=1)

    def mk(x_ref, w_ref, *o_refs):
        xb = x_ref[...]
        if sig:
            xb = jax.nn.sigmoid(xb)
        res = jnp.dot(xb, w_ref[...], preferred_element_type=jnp.float32)
        for i, o in enumerate(o_refs):
            o[...] = res[:, i * _C:(i + 1) * _C]

    outs = pl.pallas_call(
        mk,
        grid=(n // br,),
        in_specs=[pl.BlockSpec((br, _C), lambda i: (i, 0)),
                  pl.BlockSpec((_C, kdim), lambda i: (0, 0))],
        out_specs=[pl.BlockSpec((br, _C), lambda i: (i, 0))] * len(ws),
        out_shape=[jax.ShapeDtypeStruct((n, _C), jnp.float32)] * len(ws),
    )(x, w)
    return outs if len(ws) > 1 else outs[0]


def _final_linear(x, w_lin, b, br):
    n = x.shape[0]

    def mk(x_ref, w_ref, b_ref, o_ref):
        xb = jax.nn.sigmoid(x_ref[...])
        o_ref[...] = (jnp.dot(xb, w_ref[...],
                              preferred_element_type=jnp.float32)
                      + b_ref[0, 0])

    out = pl.pallas_call(
        mk,
        grid=(n // br,),
        in_specs=[pl.BlockSpec((br, _C), lambda i: (i, 0)),
                  pl.BlockSpec((_C, 1), lambda i: (0, 0)),
                  pl.BlockSpec((1, 1), lambda i: (0, 0))],
        out_specs=pl.BlockSpec((br, 1), lambda i: (i, 0)),
        out_shape=jax.ShapeDtypeStruct((n, 1), jnp.float32),
    )(x, w_lin, b.reshape(1, 1))
    return out.reshape(n)


def kernel(x0, x1, x2, b1_row, b1_col, b1_val, b2_row, b2_col, b2_val,
           a0_row, a0_col, a0_val, a1_row, a1_col, a1_val, a2_row, a2_col,
           a2_val, params):
    p = params
    N0, N1 = x0.shape[0], x1.shape[0]

    y0s, y0l = _mm(x0, [p['W_same_0_0'], p['W_lth_0_1']], sig=False, br=2000)
    y1s, y1h = _mm(x1, [p['W_same_0_1'], p['W_htl_0_0']], sig=False, br=2000)
    y2h = _mm(x2, [p['W_htl_0_1']], sig=False, br=2000)

    t0 = _pad_task(a0_row, a0_col, a0_val, N0)
    t1 = _pad_task(b1_row, b1_col, b1_val, N0)
    t2 = _pad_task(a1_row, a1_col, a1_val, N1)
    t3 = _pad_task(b2_row, b2_col, b2_val, N1)
    t4 = _pad_task(b1_col, b1_row, b1_val, N1)

    phases0 = [
        dict(CH=5000, slots=1, out=0, tasks=[(0, 0), (1, 1)]),
        dict(CH=10000, slots=8, out=1, tasks=[(2, 2), (3, 3), (4, 4)]),
    ]
    m0, m1 = _sc_call(phases0, [y0s, y1h, y1s, y2h, y0l],
                      [t0, t1, t2, t3, t4], [N0, N1], 10000)

    z0 = _mm(m0, [p['W_same_1_0']], sig=True, br=2000)
    z1 = _mm(m1, [p['W_htl_1_0']], sig=True, br=2000)

    phases1 = [dict(CH=5000, slots=1, out=0, tasks=[(0, 0), (1, 1)])]
    (m0b,) = _sc_call(phases1, [z0, z1], [t0, t1], [N0], 5000)

    return _final_linear(m0b, p['W_lin'], p['b_lin'], br=2000)
